# baseline (speedup 1.0000x reference)
import jax
import jax.numpy as jnp
from jax import lax
from jax.experimental import pallas as pl
from jax.experimental.pallas import tpu as pltpu

COMMITMENT_COST = 0.25
ROW_TILE = 16


def _vq_body(x_ref, w_ref, out_ref, dsum_ref, m_ref, oh_ref, xsq_ref,
             dmin_ref):
    i = pl.program_id(0)
    xb = x_ref[0]
    w = w_ref[...]
    k = w.shape[0]
    bn = xb.shape[0]
    m_ref[...] = lax.dot_general(xb, w, (((1,), (1,)), ((), ())))
    xsq_ref[...] = jnp.sum(xb ** 2, axis=-1, keepdims=True)
    w_sq = jnp.sum(w ** 2, axis=-1)[None, :]
    iota = lax.broadcasted_iota(jnp.int32, (ROW_TILE, k), 1).astype(
        jnp.float32)

    def tile(t, _):
        sl = pl.ds(t * ROW_TILE, ROW_TILE)
        mt = m_ref[sl, :]
        dt = xsq_ref[sl, :] - 2.0 * mt + w_sq
        dmin = jnp.min(dt, axis=1, keepdims=True)
        idx = jnp.min(jnp.where(dt == dmin, iota, jnp.float32(k)),
                      axis=1, keepdims=True)
        oh_ref[sl, :] = jnp.where(iota == idx, 1.0, 0.0).astype(jnp.bfloat16)
        dmin_ref[sl, :] = dmin
        return 0

    lax.fori_loop(0, bn // ROW_TILE, tile, 0, unroll=2)

    q = lax.dot_general(oh_ref[...], w.astype(jnp.bfloat16),
                        (((1,), (0,)), ((), ())),
                        preferred_element_type=jnp.float32)
    out_ref[0] = xb + (q - xb)

    @pl.when(i == 0)
    def _():
        dsum_ref[0, 0] = 0.0

    dsum_ref[0, 0] += jnp.sum(dmin_ref[...])

    @pl.when(i == pl.num_programs(0) - 1)
    def _():
        total = w.shape[1] * pl.num_programs(0) * bn
        dsum_ref[0, 0] = dsum_ref[0, 0] * (COMMITMENT_COST / total)


def kernel(x, W):
    b1, b2, d = x.shape
    k = W.shape[0]
    n = b1 * b2
    n_blocks = 2
    bn = n // n_blocks
    xf = x.reshape(n_blocks, bn, d)

    out, dsum = pl.pallas_call(
        _vq_body,
        grid=(n_blocks,),
        in_specs=[
            pl.BlockSpec((1, bn, d), lambda i: (i, 0, 0)),
            pl.BlockSpec((k, d), lambda i: (0, 0)),
        ],
        out_specs=[
            pl.BlockSpec((1, bn, d), lambda i: (i, 0, 0)),
            pl.BlockSpec((1, 1), lambda i: (0, 0), memory_space=pltpu.SMEM),
        ],
        out_shape=[
            jax.ShapeDtypeStruct((n_blocks, bn, d), jnp.float32),
            jax.ShapeDtypeStruct((1, 1), jnp.float32),
        ],
        scratch_shapes=[
            pltpu.VMEM((bn, k), jnp.float32),
            pltpu.VMEM((bn, k), jnp.bfloat16),
            pltpu.VMEM((bn, 1), jnp.float32),
            pltpu.VMEM((bn, 1), jnp.float32),
        ],
    )(xf, W)

    return (out.reshape(x.shape), dsum.reshape(()))


# speedup vs baseline: 2.5442x; 2.5442x over previous
import jax
import jax.numpy as jnp
from jax import lax
from jax.experimental import pallas as pl
from jax.experimental.pallas import tpu as pltpu

COMMITMENT_COST = 0.25


def _vq_body(x_ref, w_ref, out_ref, dsum_ref):
    i = pl.program_id(0)
    xb = x_ref[0]
    w = w_ref[...]
    k = w.shape[0]
    m = lax.dot_general(xb, w, (((1,), (1,)), ((), ())))
    x_sq = jnp.sum(xb ** 2, axis=-1, keepdims=True)
    w_sq = jnp.sum(w ** 2, axis=-1)[None, :]
    d = x_sq - 2.0 * m + w_sq
    dmin = jnp.min(d, axis=1, keepdims=True)
    iota = lax.broadcasted_iota(jnp.int32, d.shape, 1).astype(jnp.float32)
    idx = jnp.min(jnp.where(d == dmin, iota, jnp.float32(k)),
                  axis=1, keepdims=True)
    onehot = jnp.where(iota == idx, 1.0, 0.0).astype(jnp.bfloat16)
    q = lax.dot_general(onehot, w.astype(jnp.bfloat16),
                        (((1,), (0,)), ((), ())),
                        preferred_element_type=jnp.float32)
    out_ref[0] = xb + (q - xb)

    @pl.when(i == 0)
    def _():
        dsum_ref[0, 0] = 0.0

    dsum_ref[0, 0] += jnp.sum(dmin)

    @pl.when(i == pl.num_programs(0) - 1)
    def _():
        total = w.shape[1] * pl.num_programs(0) * xb.shape[0]
        dsum_ref[0, 0] = dsum_ref[0, 0] * (COMMITMENT_COST / total)


def kernel(x, W):
    b1, b2, d = x.shape
    k = W.shape[0]
    n = b1 * b2
    n_blocks = 2
    bn = n // n_blocks
    xf = x.reshape(n_blocks, bn, d)

    out, dsum = pl.pallas_call(
        _vq_body,
        grid=(n_blocks,),
        in_specs=[
            pl.BlockSpec((1, bn, d), lambda i: (i, 0, 0)),
            pl.BlockSpec((k, d), lambda i: (0, 0)),
        ],
        out_specs=[
            pl.BlockSpec((1, bn, d), lambda i: (i, 0, 0)),
            pl.BlockSpec((1, 1), lambda i: (0, 0), memory_space=pltpu.SMEM),
        ],
        out_shape=[
            jax.ShapeDtypeStruct((n_blocks, bn, d), jnp.float32),
            jax.ShapeDtypeStruct((1, 1), jnp.float32),
        ],
    )(xf, W)

    return (out.reshape(x.shape), dsum.reshape(()))
